# baseline (speedup 1.0000x reference)
import functools

import jax
import jax.numpy as jnp
from jax import lax
from jax.experimental import pallas as pl
from jax.experimental.pallas import tpu as pltpu
from jax.experimental.pallas import tpu_sc as plsc

N = 100000
D = 64
NC = 2
NS = 16
NCHUNK = 4
RCH = N // NCHUNK
STRIPE = 1568
WMAIN = 1480
WEXTRA = STRIPE - WMAIN
RGARB = NS * STRIPE
ACC_ROWS = RGARB + 8
K = 384
SUB = 128
NSUB = K // SUB
ZREM = STRIPE - (STRIPE // SUB) * SUB


def _ceil_batches(e):
    nb = -(-e // (NS * K))
    return nb + (nb % 2)


def _pad_edges(rows, cols, vals, e_pad):
    e = rows.shape[0]
    pad = e_pad - e
    rows = jnp.concatenate([rows, jnp.full((pad,), N, jnp.int32)])
    cols = jnp.concatenate([cols, jnp.zeros((pad,), jnp.int32)])
    vals = jnp.concatenate([vals, jnp.zeros((pad,), jnp.float32)])
    return rows, cols, vals


def _sc_body(nb_a, nb_b,
             ego, r_a, c_a, v_a, r_b, c_b, v_b,
             o_s, o_li,
             acc, gbuf,
             rowbuf0, colbuf0, valbuf0, destbuf0,
             rowbuf1, colbuf1, valbuf1, destbuf1,
             cdest, ccol, cval,
             isem, ssem, gsems):
    c = lax.axis_index("c")
    s = lax.axis_index("s")

    zero16 = jnp.zeros((16,), jnp.float32)
    garb16 = jnp.full((16,), RGARB, jnp.int32)
    zeroi16 = jnp.zeros((16,), jnp.int32)
    bufs = ((rowbuf0, colbuf0, valbuf0, destbuf0),
            (rowbuf1, colbuf1, valbuf1, destbuf1))

    for db in (destbuf0, destbuf1):
        for i in range(NSUB):
            def _gi(j, cc, db=db, i=i):
                db[i, pl.ds(j * 16, 16)] = garb16
                return cc
            lax.fori_loop(0, SUB // 16, _gi, 0)

    def _fire_idx(rows_h, cols_h, vals_h, base, rb, cb, vb):
        pltpu.async_copy(rows_h.at[pl.ds(base, K)], rb, isem)
        pltpu.async_copy(cols_h.at[pl.ds(base, K)], cb, isem)
        pltpu.async_copy(vals_h.at[pl.ds(base, K)], vb, isem)

    def _wait_idx(rows_h, vals_h, rb, cb, vb):
        pltpu.make_async_copy(rows_h.at[pl.ds(0, K)], rb, isem).wait()
        pltpu.make_async_copy(rows_h.at[pl.ds(0, K)], cb, isem).wait()
        pltpu.make_async_copy(vals_h.at[pl.ds(0, K)], vb, isem).wait()

    def _wait_scatters(db, nprev):
        for i in range(NSUB):
            @pl.when(i * SUB < nprev)
            def _(i=i):
                pltpu.make_async_copy(gbuf.at[i], acc.at[db.at[i]],
                                      ssem).wait()

    def one_pass(rows_h, cols_h, vals_h, out_h, nbatches, chunk):
        lo = chunk * RCH
        hi = lo + RCH

        for i in range(NSUB):
            def _zb(r, cc, i=i):
                for h in range(D // 16):
                    gbuf[i, r, pl.ds(h * 16, 16)] = zero16
                return cc
            lax.fori_loop(0, SUB, _zb, 0)

        def zeroq(q, carry):
            pltpu.sync_copy(gbuf.at[0],
                            acc.at[pl.ds(s * STRIPE + q * SUB, SUB)])
            return carry
        lax.fori_loop(0, STRIPE // SUB, zeroq, 0)
        pltpu.sync_copy(gbuf.at[0, pl.ds(0, ZREM)],
                        acc.at[pl.ds(s * STRIPE + (STRIPE // SUB) * SUB, ZREM)])
        plsc.subcore_barrier()

        tile_base = s * (nbatches * K)

        for i in range(NSUB):
            pltpu.async_copy(gbuf.at[i], acc.at[destbuf0.at[i]], ssem,
                             add=True)
        _fire_idx(rows_h, cols_h, vals_h, tile_base,
                  rowbuf0, colbuf0, valbuf0)

        def batch(b, nprev, cur, nxt):
            rb, cb, vb, db = cur
            nrb, ncb, nvb, ndb = nxt
            _wait_idx(rows_h, vals_h, rb, cb, vb)
            bn = jnp.minimum(b + 1, nbatches - 1)
            _fire_idx(rows_h, cols_h, vals_h, tile_base + bn * K,
                      nrb, ncb, nvb)

            def _clr(j, cc):
                cdest[pl.ds(j * 16, 16)] = garb16
                ccol[pl.ds(j * 16, 16)] = zeroi16
                cval[pl.ds(j * 16, 16)] = zero16
                return cc
            lax.fori_loop(0, (K + 16) // 16, _clr, 0)

            def compact(j, np16):
                r = rb[pl.ds(j * 16, 16)]
                v = vb[pl.ds(j * 16, 16)]
                cc_ = cb[pl.ds(j * 16, 16)]
                m = (r >= lo) & (r < hi)
                mi = m.astype(jnp.int32)
                pos = np16 + plsc.cumsum(mi) - 1
                plsc.store_scatter(cdest, [pos], r - lo, mask=m)
                plsc.store_scatter(ccol, [pos], cc_, mask=m)
                plsc.store_scatter(cval, [pos], v, mask=m)
                return np16 + plsc.all_reduce_population_count(m)
            n16 = lax.fori_loop(0, K // 16, compact,
                                jnp.zeros((16,), jnp.int32))
            n = n16[0]

            _wait_scatters(ndb, nprev)

            for i in range(NSUB):
                @pl.when(i * SUB < n)
                def _(i=i):
                    def _cp(j, cc):
                        db[i, pl.ds(j * 16, 16)] = \
                            cdest[pl.ds(i * SUB + j * 16, 16)]
                        return cc
                    lax.fori_loop(0, SUB // 16, _cp, 0)

            for i in range(NSUB):
                @pl.when(i * SUB < n)
                def _(i=i):
                    pltpu.async_copy(ego.at[ccol.at[pl.ds(i * SUB, SUB)]],
                                     gbuf.at[i], gsems.at[i])

            for i in range(NSUB):
                @pl.when(i * SUB < n)
                def _(i=i):
                    pltpu.make_async_copy(
                        ego.at[ccol.at[pl.ds(i * SUB, SUB)]],
                        gbuf.at[i], gsems.at[i]).wait()

                    def scale(j, cc):
                        vk = cval[pl.ds(i * SUB + j * 16, 16)]
                        for k in range(16):
                            e = j * 16 + k
                            sv = vk[k]
                            for h in range(D // 16):
                                gbuf[i, e, pl.ds(h * 16, 16)] = \
                                    gbuf[i, e, pl.ds(h * 16, 16)] * sv
                        return cc
                    lax.fori_loop(0, SUB // 16, scale, 0)
                    pltpu.async_copy(gbuf.at[i], acc.at[db.at[i]], ssem,
                                     add=True)
            return n

        def pair(p, nprev):
            n0 = batch(2 * p, nprev, bufs[0], bufs[1])
            n1 = batch(2 * p + 1, n0, bufs[1], bufs[0])
            return n1

        nlast = lax.fori_loop(0, nbatches // 2, pair, jnp.int32(NSUB * SUB))
        _wait_scatters(destbuf1, nlast)
        _wait_idx(rows_h, vals_h, rowbuf0, colbuf0, valbuf0)
        plsc.subcore_barrier()

        pltpu.sync_copy(acc.at[pl.ds(s * STRIPE, WMAIN)],
                        out_h.at[pl.ds(lo + s * STRIPE, WMAIN)])

        @pl.when(s < NS - 1)
        def _():
            pltpu.sync_copy(acc.at[pl.ds(s * STRIPE + WMAIN, WEXTRA)],
                            out_h.at[pl.ds(lo + s * STRIPE + WMAIN, WEXTRA)])

        plsc.subcore_barrier()

    one_pass(r_a, c_a, v_a, o_s, nb_a, 2 * c)
    one_pass(r_a, c_a, v_a, o_s, nb_a, 2 * c + 1)
    one_pass(r_b, c_b, v_b, o_li, nb_b, 2 * c)
    one_pass(r_b, c_b, v_b, o_li, nb_b, 2 * c + 1)


def _sc_spmm(ego, r_a, c_a, v_a, r_b, c_b, v_b, nb_a, nb_b):
    mesh = plsc.VectorSubcoreMesh(core_axis_name="c", subcore_axis_name="s",
                                  num_cores=NC, num_subcores=NS)
    out = jax.ShapeDtypeStruct((N, D), jnp.float32)
    f = pl.kernel(
        functools.partial(_sc_body, nb_a, nb_b),
        out_type=(out, out),
        mesh=mesh,
        scratch_types=[
            pltpu.VMEM_SHARED((ACC_ROWS, D), jnp.float32),
            pltpu.VMEM((NSUB, SUB, D), jnp.float32),
            pltpu.VMEM((K,), jnp.int32),
            pltpu.VMEM((K,), jnp.int32),
            pltpu.VMEM((K,), jnp.float32),
            pltpu.VMEM((NSUB, SUB), jnp.int32),
            pltpu.VMEM((K,), jnp.int32),
            pltpu.VMEM((K,), jnp.int32),
            pltpu.VMEM((K,), jnp.float32),
            pltpu.VMEM((NSUB, SUB), jnp.int32),
            pltpu.VMEM((K + 16,), jnp.int32),
            pltpu.VMEM((K + 16,), jnp.int32),
            pltpu.VMEM((K + 16,), jnp.float32),
            pltpu.SemaphoreType.DMA,
            pltpu.SemaphoreType.DMA,
            pltpu.SemaphoreType.DMA((NSUB,)),
        ],
        compiler_params=pltpu.CompilerParams(use_tc_tiling_on_sc=False,
                                            needs_layout_passes=False),
        name="sc_coo_spmm",
    )
    return f(ego, r_a, c_a, v_a, r_b, c_b, v_b)


def _tc_body(side, sli, ego, w1, b1, w2, b2, out):
    xint = side[...] * ego[...]
    y = (lax.dot_general(sli[...], w1[...], (((1,), (1,)), ((), ())),
                         preferred_element_type=jnp.float32)
         + lax.dot_general(xint, w2[...], (((1,), (1,)), ((), ())),
                           preferred_element_type=jnp.float32)
         + b1[...] + b2[...])
    out[...] = jnp.where(y >= 0, y, 0.01 * y)


def _tc_dense(side, sli, ego, W1, b1, W2, b2):
    BR = 1000
    grid = (N // BR,)
    full = pl.BlockSpec((BR, D), lambda i: (i, 0))
    wspec = pl.BlockSpec((D, D), lambda i: (0, 0))
    bspec = pl.BlockSpec((1, D), lambda i: (0, 0))
    return pl.pallas_call(
        _tc_body,
        grid=grid,
        in_specs=[full, full, full, wspec, bspec, wspec, bspec],
        out_specs=full,
        out_shape=jax.ShapeDtypeStruct((N, D), jnp.float32),
    )(side, sli, ego, W1, b1.reshape(1, D), W2, b2.reshape(1, D))


def kernel(ego_embeddings, a_in_indices, a_in_values, a_in_plusI_indices,
           a_in_plusI_values, W1, b1, W2, b2):
    nb_a = _ceil_batches(a_in_values.shape[0])
    nb_b = _ceil_batches(a_in_plusI_values.shape[0])
    r_a, c_a, v_a = _pad_edges(a_in_indices[0], a_in_indices[1], a_in_values,
                               nb_a * NS * K)
    r_b, c_b, v_b = _pad_edges(a_in_plusI_indices[0], a_in_plusI_indices[1],
                               a_in_plusI_values, nb_b * NS * K)

    side, sli = _sc_spmm(ego_embeddings, r_a, c_a, v_a, r_b, c_b, v_b,
                         nb_a, nb_b)
    return _tc_dense(side, sli, ego_embeddings, W1, b1, W2, b2)


# speedup vs baseline: 2.4057x; 2.4057x over previous
import functools

import jax
import jax.numpy as jnp
from jax import lax
from jax.experimental import pallas as pl
from jax.experimental.pallas import tpu as pltpu
from jax.experimental.pallas import tpu_sc as plsc

N = 100000
D = 64
NC = 2
NS = 16
NCHUNK = 4
RCH = N // NCHUNK
STRIPE = 1568
WMAIN = 1480
WEXTRA = STRIPE - WMAIN
RGARB = NS * STRIPE
ACC_ROWS = RGARB + 8
K = 384
SUB = 128
NSUB = K // SUB
ZREM = STRIPE - (STRIPE // SUB) * SUB


def _ceil_batches(e):
    nb = -(-e // (NS * K))
    return nb + (nb % 2)


def _pad_edges(rows, cols, vals, e_pad):
    e = rows.shape[0]
    pad = e_pad - e
    rows = jnp.concatenate([rows, jnp.full((pad,), N, jnp.int32)])
    cols = jnp.concatenate([cols, jnp.zeros((pad,), jnp.int32)])
    vals = jnp.concatenate([vals, jnp.zeros((pad,), jnp.float32)])
    return rows, cols, vals


def _sc_body(nb_a, nb_b,
             ego, r_a, c_a, v_a, r_b, c_b, v_b,
             o_s, o_li,
             acc, gbuf,
             rowbuf0, colbuf0, valbuf0, destbuf0,
             rowbuf1, colbuf1, valbuf1, destbuf1,
             isem, ssem, gsems):
    c = lax.axis_index("c")
    s = lax.axis_index("s")

    zero16 = jnp.zeros((16,), jnp.float32)
    garb16 = jnp.full((16,), RGARB, jnp.int32)
    zeroi16 = jnp.zeros((16,), jnp.int32)
    bufs = ((rowbuf0, colbuf0, valbuf0, destbuf0),
            (rowbuf1, colbuf1, valbuf1, destbuf1))

    for db in (destbuf0, destbuf1):
        for i in range(NSUB):
            def _gi(j, cc, db=db, i=i):
                db[i, pl.ds(j * 16, 16)] = garb16
                return cc
            lax.fori_loop(0, SUB // 16, _gi, 0)

    def _fire_idx(rows_h, cols_h, vals_h, base, rb, cb, vb):
        pltpu.async_copy(rows_h.at[pl.ds(base, K)], rb, isem)
        pltpu.async_copy(cols_h.at[pl.ds(base, K)], cb, isem)
        pltpu.async_copy(vals_h.at[pl.ds(base, K)], vb, isem)

    def _wait_idx(rows_h, vals_h, rb, cb, vb):
        pltpu.make_async_copy(rows_h.at[pl.ds(0, K)], rb, isem).wait()
        pltpu.make_async_copy(rows_h.at[pl.ds(0, K)], cb, isem).wait()
        pltpu.make_async_copy(vals_h.at[pl.ds(0, K)], vb, isem).wait()

    def _wait_scatters(db, nprev):
        del nprev
        for i in range(NSUB):
            pltpu.make_async_copy(gbuf.at[i], acc.at[db.at[i]], ssem).wait()

    def one_pass(rows_h, cols_h, vals_h, out_h, nbatches, chunk):
        lo = chunk * RCH
        hi = lo + RCH

        for i in range(NSUB):
            def _zb(r, cc, i=i):
                for h in range(D // 16):
                    gbuf[i, r, pl.ds(h * 16, 16)] = zero16
                return cc
            lax.fori_loop(0, SUB, _zb, 0)

        def zeroq(q, carry):
            pltpu.sync_copy(gbuf.at[0],
                            acc.at[pl.ds(s * STRIPE + q * SUB, SUB)])
            return carry
        lax.fori_loop(0, STRIPE // SUB, zeroq, 0)
        pltpu.sync_copy(gbuf.at[0, pl.ds(0, ZREM)],
                        acc.at[pl.ds(s * STRIPE + (STRIPE // SUB) * SUB, ZREM)])
        plsc.subcore_barrier()

        tile_base = s * (nbatches * K)

        for i in range(NSUB):
            pltpu.async_copy(gbuf.at[i], acc.at[destbuf0.at[i]], ssem,
                             add=True)
        _fire_idx(rows_h, cols_h, vals_h, tile_base,
                  rowbuf0, colbuf0, valbuf0)

        def batch(b, nprev, cur, nxt):
            rb, cb, vb, db = cur
            nrb, ncb, nvb, ndb = nxt
            _wait_idx(rows_h, vals_h, rb, cb, vb)
            bn = jnp.minimum(b + 1, nbatches - 1)
            _fire_idx(rows_h, cols_h, vals_h, tile_base + bn * K,
                      nrb, ncb, nvb)

            for i in range(NSUB):
                def prep(j, cc, i=i):
                    off = i * SUB + j * 16
                    r = rb[pl.ds(off, 16)]
                    v = vb[pl.ds(off, 16)]
                    m = (r >= lo) & (r < hi)
                    db[i, pl.ds(j * 16, 16)] = jnp.where(m, r - lo, RGARB)
                    vb[pl.ds(off, 16)] = jnp.where(m, v, 0.0)
                    return cc
                lax.fori_loop(0, SUB // 16, prep, 0)

            _wait_scatters(ndb, nprev)

            cps = [
                pltpu.async_copy(ego.at[cb.at[pl.ds(i * SUB, SUB)]],
                                 gbuf.at[i], gsems.at[i])
                for i in range(NSUB)
            ]
            for i in range(NSUB):
                cps[i].wait()

                def scale(j, cc, i=i):
                    vk = vb[pl.ds(i * SUB + j * 16, 16)]
                    for k in range(16):
                        e = j * 16 + k
                        sv = vk[k]
                        for h in range(D // 16):
                            gbuf[i, e, pl.ds(h * 16, 16)] = \
                                gbuf[i, e, pl.ds(h * 16, 16)] * sv
                    return cc
                lax.fori_loop(0, SUB // 16, scale, 0)
                pltpu.async_copy(gbuf.at[i], acc.at[db.at[i]], ssem,
                                 add=True)
            return jnp.int32(NSUB * SUB)

        def pair(p, nprev):
            n0 = batch(2 * p, nprev, bufs[0], bufs[1])
            n1 = batch(2 * p + 1, n0, bufs[1], bufs[0])
            return n1

        nlast = lax.fori_loop(0, nbatches // 2, pair, jnp.int32(NSUB * SUB))
        _wait_scatters(destbuf1, nlast)
        _wait_idx(rows_h, vals_h, rowbuf0, colbuf0, valbuf0)
        plsc.subcore_barrier()

        pltpu.sync_copy(acc.at[pl.ds(s * STRIPE, WMAIN)],
                        out_h.at[pl.ds(lo + s * STRIPE, WMAIN)])

        @pl.when(s < NS - 1)
        def _():
            pltpu.sync_copy(acc.at[pl.ds(s * STRIPE + WMAIN, WEXTRA)],
                            out_h.at[pl.ds(lo + s * STRIPE + WMAIN, WEXTRA)])

        plsc.subcore_barrier()

    one_pass(r_a, c_a, v_a, o_s, nb_a, 2 * c)
    one_pass(r_a, c_a, v_a, o_s, nb_a, 2 * c + 1)
    one_pass(r_b, c_b, v_b, o_li, nb_b, 2 * c)
    one_pass(r_b, c_b, v_b, o_li, nb_b, 2 * c + 1)


def _sc_spmm(ego, r_a, c_a, v_a, r_b, c_b, v_b, nb_a, nb_b):
    mesh = plsc.VectorSubcoreMesh(core_axis_name="c", subcore_axis_name="s",
                                  num_cores=NC, num_subcores=NS)
    out = jax.ShapeDtypeStruct((N, D), jnp.float32)
    f = pl.kernel(
        functools.partial(_sc_body, nb_a, nb_b),
        out_type=(out, out),
        mesh=mesh,
        scratch_types=[
            pltpu.VMEM_SHARED((ACC_ROWS, D), jnp.float32),
            pltpu.VMEM((NSUB, SUB, D), jnp.float32),
            pltpu.VMEM((K,), jnp.int32),
            pltpu.VMEM((K,), jnp.int32),
            pltpu.VMEM((K,), jnp.float32),
            pltpu.VMEM((NSUB, SUB), jnp.int32),
            pltpu.VMEM((K,), jnp.int32),
            pltpu.VMEM((K,), jnp.int32),
            pltpu.VMEM((K,), jnp.float32),
            pltpu.VMEM((NSUB, SUB), jnp.int32),
            pltpu.SemaphoreType.DMA,
            pltpu.SemaphoreType.DMA,
            pltpu.SemaphoreType.DMA((NSUB,)),
        ],
        compiler_params=pltpu.CompilerParams(use_tc_tiling_on_sc=False,
                                            needs_layout_passes=False),
        name="sc_coo_spmm",
    )
    return f(ego, r_a, c_a, v_a, r_b, c_b, v_b)


def _tc_body(side, sli, ego, w1, b1, w2, b2, out):
    xint = side[...] * ego[...]
    y = (lax.dot_general(sli[...], w1[...], (((1,), (1,)), ((), ())),
                         preferred_element_type=jnp.float32)
         + lax.dot_general(xint, w2[...], (((1,), (1,)), ((), ())),
                           preferred_element_type=jnp.float32)
         + b1[...] + b2[...])
    out[...] = jnp.where(y >= 0, y, 0.01 * y)


def _tc_dense(side, sli, ego, W1, b1, W2, b2):
    BR = 1000
    grid = (N // BR,)
    full = pl.BlockSpec((BR, D), lambda i: (i, 0))
    wspec = pl.BlockSpec((D, D), lambda i: (0, 0))
    bspec = pl.BlockSpec((1, D), lambda i: (0, 0))
    return pl.pallas_call(
        _tc_body,
        grid=grid,
        in_specs=[full, full, full, wspec, bspec, wspec, bspec],
        out_specs=full,
        out_shape=jax.ShapeDtypeStruct((N, D), jnp.float32),
    )(side, sli, ego, W1, b1.reshape(1, D), W2, b2.reshape(1, D))


def kernel(ego_embeddings, a_in_indices, a_in_values, a_in_plusI_indices,
           a_in_plusI_values, W1, b1, W2, b2):
    nb_a = _ceil_batches(a_in_values.shape[0])
    nb_b = _ceil_batches(a_in_plusI_values.shape[0])
    r_a, c_a, v_a = _pad_edges(a_in_indices[0], a_in_indices[1], a_in_values,
                               nb_a * NS * K)
    r_b, c_b, v_b = _pad_edges(a_in_plusI_indices[0], a_in_plusI_indices[1],
                               a_in_plusI_values, nb_b * NS * K)

    side, sli = _sc_spmm(ego_embeddings, r_a, c_a, v_a, r_b, c_b, v_b,
                         nb_a, nb_b)
    return _tc_dense(side, sli, ego_embeddings, W1, b1, W2, b2)


# speedup vs baseline: 4.0501x; 1.6836x over previous
import functools

import jax
import jax.numpy as jnp
from jax import lax
from jax.experimental import pallas as pl
from jax.experimental.pallas import tpu as pltpu
from jax.experimental.pallas import tpu_sc as plsc

N = 100000
D = 64
DH = 32
NC = 2
NS = 16
RPC = N // NC
STRIPE = 3128
WMAIN = 3080
WEXTRA = STRIPE - WMAIN
RGARB = NS * STRIPE
ACC_ROWS = RGARB + 8
K = 512
SUB = 128
NSUB = K // SUB
ZREM = STRIPE - (STRIPE // SUB) * SUB


def _ceil_batches(e):
    nb = -(-e // (NS * K))
    return nb + (nb % 2)


def _pad_edges(rows, cols, vals, e_pad):
    e = rows.shape[0]
    pad = e_pad - e
    rows = jnp.concatenate([rows, jnp.full((pad,), N, jnp.int32)])
    cols = jnp.concatenate([cols, jnp.zeros((pad,), jnp.int32)])
    vals = jnp.concatenate([vals, jnp.zeros((pad,), jnp.float32)])
    return rows, cols, vals


def _shuffle_half_bf16(e):
    es = jnp.stack([e[:, :16], e[:, 16:]], axis=2).reshape(N, 32)
    return es.astype(jnp.bfloat16)


def _sc_body(nb_a, nb_b,
             ego_lo, ego_hi, r_a, c_a, v_a, r_b, c_b, v_b,
             o_s_lo, o_s_hi, o_li_lo, o_li_hi,
             acc, gybuf, gsbuf,
             rowbuf0, colbuf0, valbuf0, destbuf0,
             rowbuf1, colbuf1, valbuf1, destbuf1,
             isem, ssem, gsems):
    c = lax.axis_index("c")
    s = lax.axis_index("s")
    lo = c * RPC
    hi = lo + RPC

    zero16 = jnp.zeros((16,), jnp.float32)
    garb16 = jnp.full((16,), RGARB, jnp.int32)
    himask = jnp.full((16,), -65536, jnp.int32)
    bufs = ((rowbuf0, colbuf0, valbuf0, destbuf0),
            (rowbuf1, colbuf1, valbuf1, destbuf1))

    for db in (destbuf0, destbuf1):
        for i in range(NSUB):
            def _gi(j, cc, db=db, i=i):
                db[i, pl.ds(j * 16, 16)] = garb16
                return cc
            lax.fori_loop(0, SUB // 16, _gi, 0)

    def _fire_idx(rows_h, cols_h, vals_h, base, rb, cb, vb):
        pltpu.async_copy(rows_h.at[pl.ds(base, K)], rb, isem)
        pltpu.async_copy(cols_h.at[pl.ds(base, K)], cb, isem)
        pltpu.async_copy(vals_h.at[pl.ds(base, K)], vb, isem)

    def _wait_idx(rows_h, vals_h, rb, cb, vb):
        pltpu.make_async_copy(rows_h.at[pl.ds(0, K)], rb, isem).wait()
        pltpu.make_async_copy(rows_h.at[pl.ds(0, K)], cb, isem).wait()
        pltpu.make_async_copy(vals_h.at[pl.ds(0, K)], vb, isem).wait()

    def _wait_scatters(db):
        for i in range(NSUB):
            pltpu.make_async_copy(gsbuf.at[i], acc.at[db.at[i]], ssem).wait()

    def one_pass(rows_h, cols_h, vals_h, ego_h, out_h, nbatches):
        for i in range(NSUB):
            def _zb(r, cc, i=i):
                gsbuf[i, r, pl.ds(0, 16)] = zero16
                gsbuf[i, r, pl.ds(16, 16)] = zero16
                return cc
            lax.fori_loop(0, SUB, _zb, 0)

        def zeroq(q, carry):
            pltpu.sync_copy(gsbuf.at[0],
                            acc.at[pl.ds(s * STRIPE + q * SUB, SUB)])
            return carry
        lax.fori_loop(0, STRIPE // SUB, zeroq, 0)
        pltpu.sync_copy(gsbuf.at[0, pl.ds(0, ZREM)],
                        acc.at[pl.ds(s * STRIPE + (STRIPE // SUB) * SUB, ZREM)])
        plsc.subcore_barrier()

        tile_base = s * (nbatches * K)

        for i in range(NSUB):
            pltpu.async_copy(gsbuf.at[i], acc.at[destbuf0.at[i]], ssem,
                             add=True)
        _fire_idx(rows_h, cols_h, vals_h, tile_base,
                  rowbuf0, colbuf0, valbuf0)

        def batch(b, cur, nxt):
            rb, cb, vb, db = cur
            nrb, ncb, nvb, ndb = nxt
            _wait_idx(rows_h, vals_h, rb, cb, vb)
            bn = jnp.minimum(b + 1, nbatches - 1)
            _fire_idx(rows_h, cols_h, vals_h, tile_base + bn * K,
                      nrb, ncb, nvb)

            for i in range(NSUB):
                def prep(j, cc, i=i):
                    off = i * SUB + j * 16
                    r = rb[pl.ds(off, 16)]
                    v = vb[pl.ds(off, 16)]
                    m = (r >= lo) & (r < hi)
                    db[i, pl.ds(j * 16, 16)] = jnp.where(m, r - lo, RGARB)
                    vb[pl.ds(off, 16)] = jnp.where(m, v, 0.0)
                    return cc
                lax.fori_loop(0, SUB // 16, prep, 0)

            cps = [
                pltpu.async_copy(ego_h.at[cb.at[pl.ds(i * SUB, SUB)]],
                                 gybuf.at[i], gsems.at[i])
                for i in range(NSUB)
            ]
            _wait_scatters(ndb)

            for i in range(NSUB):
                cps[i].wait()

                def scale(j, cc, i=i):
                    for k in range(4):
                        e = j * 4 + k
                        sv = plsc.load_gather(
                            vb, [jnp.full((16,), i * SUB + e, jnp.int32)])
                        xi = plsc.bitcast(gybuf[i, e, pl.ds(0, 32)],
                                          jnp.int32)
                        lof = plsc.bitcast(xi << 16, jnp.float32)
                        hif = plsc.bitcast(xi & himask, jnp.float32)
                        gsbuf[i, e, pl.ds(0, 16)] = lof * sv
                        gsbuf[i, e, pl.ds(16, 16)] = hif * sv
                    return cc
                lax.fori_loop(0, SUB // 4, scale, 0)
                pltpu.async_copy(gsbuf.at[i], acc.at[db.at[i]], ssem,
                                 add=True)
            return None

        def pair(p, carry):
            batch(2 * p, bufs[0], bufs[1])
            batch(2 * p + 1, bufs[1], bufs[0])
            return carry

        lax.fori_loop(0, nbatches // 2, pair, 0)
        _wait_scatters(destbuf1)
        _wait_idx(rows_h, vals_h, rowbuf0, colbuf0, valbuf0)
        plsc.subcore_barrier()

        pltpu.sync_copy(acc.at[pl.ds(s * STRIPE, WMAIN)],
                        out_h.at[pl.ds(lo + s * STRIPE, WMAIN)])

        @pl.when(s < NS - 1)
        def _():
            pltpu.sync_copy(acc.at[pl.ds(s * STRIPE + WMAIN, WEXTRA)],
                            out_h.at[pl.ds(lo + s * STRIPE + WMAIN, WEXTRA)])

        plsc.subcore_barrier()

    one_pass(r_a, c_a, v_a, ego_lo, o_s_lo, nb_a)
    one_pass(r_a, c_a, v_a, ego_hi, o_s_hi, nb_a)
    one_pass(r_b, c_b, v_b, ego_lo, o_li_lo, nb_b)
    one_pass(r_b, c_b, v_b, ego_hi, o_li_hi, nb_b)


def _sc_spmm(ego_lo, ego_hi, r_a, c_a, v_a, r_b, c_b, v_b, nb_a, nb_b):
    mesh = plsc.VectorSubcoreMesh(core_axis_name="c", subcore_axis_name="s",
                                  num_cores=NC, num_subcores=NS)
    out = jax.ShapeDtypeStruct((N, DH), jnp.float32)
    f = pl.kernel(
        functools.partial(_sc_body, nb_a, nb_b),
        out_type=(out, out, out, out),
        mesh=mesh,
        scratch_types=[
            pltpu.VMEM_SHARED((ACC_ROWS, DH), jnp.float32),
            pltpu.VMEM((NSUB, SUB, DH), jnp.bfloat16),
            pltpu.VMEM((NSUB, SUB, DH), jnp.float32),
            pltpu.VMEM((K,), jnp.int32),
            pltpu.VMEM((K,), jnp.int32),
            pltpu.VMEM((K,), jnp.float32),
            pltpu.VMEM((NSUB, SUB), jnp.int32),
            pltpu.VMEM((K,), jnp.int32),
            pltpu.VMEM((K,), jnp.int32),
            pltpu.VMEM((K,), jnp.float32),
            pltpu.VMEM((NSUB, SUB), jnp.int32),
            pltpu.SemaphoreType.DMA,
            pltpu.SemaphoreType.DMA,
            pltpu.SemaphoreType.DMA((NSUB,)),
        ],
        compiler_params=pltpu.CompilerParams(use_tc_tiling_on_sc=False,
                                             needs_layout_passes=False),
        name="sc_coo_spmm",
    )
    return f(ego_lo, ego_hi, r_a, c_a, v_a, r_b, c_b, v_b)


def _tc_body(sl, sh, ll, lh, ego, w1, b1, w2, b2, out):
    xli = jnp.concatenate([ll[...], lh[...]], axis=1)
    xint = jnp.concatenate([sl[...], sh[...]], axis=1) * ego[...]
    y = (lax.dot_general(xli, w1[...], (((1,), (1,)), ((), ())),
                         preferred_element_type=jnp.float32)
         + lax.dot_general(xint, w2[...], (((1,), (1,)), ((), ())),
                           preferred_element_type=jnp.float32)
         + b1[...] + b2[...])
    out[...] = jnp.where(y >= 0, y, 0.01 * y)


def _tc_dense(s_lo, s_hi, li_lo, li_hi, ego, W1, b1, W2, b2):
    BR = 1000
    grid = (N // BR,)
    half = pl.BlockSpec((BR, DH), lambda i: (i, 0))
    full = pl.BlockSpec((BR, D), lambda i: (i, 0))
    wspec = pl.BlockSpec((D, D), lambda i: (0, 0))
    bspec = pl.BlockSpec((1, D), lambda i: (0, 0))
    return pl.pallas_call(
        _tc_body,
        grid=grid,
        in_specs=[half, half, half, half, full, wspec, bspec, wspec, bspec],
        out_specs=full,
        out_shape=jax.ShapeDtypeStruct((N, D), jnp.float32),
    )(s_lo, s_hi, li_lo, li_hi, ego,
      W1, b1.reshape(1, D), W2, b2.reshape(1, D))


def kernel(ego_embeddings, a_in_indices, a_in_values, a_in_plusI_indices,
           a_in_plusI_values, W1, b1, W2, b2):
    ego_lo = _shuffle_half_bf16(ego_embeddings[:, :DH])
    ego_hi = _shuffle_half_bf16(ego_embeddings[:, DH:])

    nb_a = _ceil_batches(a_in_values.shape[0])
    nb_b = _ceil_batches(a_in_plusI_values.shape[0])
    r_a, c_a, v_a = _pad_edges(a_in_indices[0], a_in_indices[1], a_in_values,
                               nb_a * NS * K)
    r_b, c_b, v_b = _pad_edges(a_in_plusI_indices[0], a_in_plusI_indices[1],
                               a_in_plusI_values, nb_b * NS * K)

    s_lo, s_hi, li_lo, li_hi = _sc_spmm(ego_lo, ego_hi,
                                        r_a, c_a, v_a, r_b, c_b, v_b,
                                        nb_a, nb_b)
    return _tc_dense(s_lo, s_hi, li_lo, li_hi, ego_embeddings, W1, b1, W2, b2)
